# baseline (speedup 1.0000x reference)
import functools

import jax
import jax.numpy as jnp
from jax import lax
from jax.experimental import pallas as pl
from jax.experimental.pallas import tpu as pltpu
from jax.experimental.pallas import tpu_sc as plsc

B = 16384
D = 64
NC = 2
NS = 16
L = 16
NW = NC * NS
BPW = B // NW
CH = 128
NCH = BPW // CH
G = BPW // L


def _body(uid_h, pid_h, nid_h, ut_h, it_h, pos_h, neg_h,
          idx_u, idx_p, idx_n, u_rows, p_rows, n_rows, pos_v, neg_v, sem):
    cid = lax.axis_index("c")
    sid = lax.axis_index("s")
    wid = sid * NC + cid
    base = wid * BPW

    pltpu.sync_copy(uid_h.at[pl.ds(base, BPW)], idx_u)
    pltpu.sync_copy(pid_h.at[pl.ds(base, BPW)], idx_p)
    pltpu.sync_copy(nid_h.at[pl.ds(base, BPW)], idx_n)

    copies = []
    for j in range(NCH):
        sl = pl.ds(j * CH, CH)
        copies.append(pltpu.async_copy(ut_h.at[idx_u.at[sl]], u_rows.at[sl], sem))
        copies.append(pltpu.async_copy(it_h.at[idx_p.at[sl]], p_rows.at[sl], sem))
        copies.append(pltpu.async_copy(it_h.at[idx_n.at[sl]], n_rows.at[sl], sem))
    for c in copies:
        c.wait()

    lane = lax.iota(jnp.int32, L)

    def group(g, carry):
        rvec = lane + g * L
        accp = jnp.zeros((L,), jnp.float32)
        accn = jnp.zeros((L,), jnp.float32)
        for d in range(D):
            cvec = jnp.full((L,), d, jnp.int32)
            u = plsc.load_gather(u_rows, [rvec, cvec])
            p = plsc.load_gather(p_rows, [rvec, cvec])
            n = plsc.load_gather(n_rows, [rvec, cvec])
            accp = accp + u * p
            accn = accn + u * n
        off = pl.multiple_of(g * L, L)
        pos_v[pl.ds(off, L)] = accp
        neg_v[pl.ds(off, L)] = accn
        return carry

    lax.fori_loop(0, G, group, 0)

    pltpu.sync_copy(pos_v, pos_h.at[pl.ds(base, BPW)])
    pltpu.sync_copy(neg_v, neg_h.at[pl.ds(base, BPW)])


@functools.partial(jax.jit, static_argnums=())
def kernel(user_ids, pos_item_ids, neg_item_ids, user_table, item_table):
    mesh = plsc.VectorSubcoreMesh(core_axis_name="c", subcore_axis_name="s")
    f = functools.partial(
        pl.kernel,
        mesh=mesh,
        compiler_params=pltpu.CompilerParams(
            needs_layout_passes=False, use_tc_tiling_on_sc=False
        ),
        out_type=(
            jax.ShapeDtypeStruct((B,), jnp.float32),
            jax.ShapeDtypeStruct((B,), jnp.float32),
        ),
        scratch_types=[
            pltpu.VMEM((BPW,), jnp.int32),
            pltpu.VMEM((BPW,), jnp.int32),
            pltpu.VMEM((BPW,), jnp.int32),
            pltpu.VMEM((BPW, D), jnp.float32),
            pltpu.VMEM((BPW, D), jnp.float32),
            pltpu.VMEM((BPW, D), jnp.float32),
            pltpu.VMEM((BPW,), jnp.float32),
            pltpu.VMEM((BPW,), jnp.float32),
            pltpu.SemaphoreType.DMA,
        ],
    )(_body)
    return f(
        user_ids.astype(jnp.int32),
        pos_item_ids.astype(jnp.int32),
        neg_item_ids.astype(jnp.int32),
        user_table,
        item_table,
    )


# speedup vs baseline: 1.5286x; 1.5286x over previous
import functools

import jax
import jax.numpy as jnp
from jax import lax
from jax.experimental import pallas as pl
from jax.experimental.pallas import tpu as pltpu
from jax.experimental.pallas import tpu_sc as plsc

B = 16384
D = 64
NC = 2
NS = 16
L = 16
NW = NC * NS
BPW = B // NW
CH = 128
NCH = BPW // CH
GPC = CH // L


def _body(uid_h, pid_h, nid_h, ut_h, it_h, pos_h, neg_h,
          idx_u, idx_p, idx_n, u_rows, p_rows, n_rows, pos_v, neg_v, sem):
    cid = lax.axis_index("c")
    sid = lax.axis_index("s")
    wid = sid * NC + cid
    base = wid * BPW

    pltpu.sync_copy(uid_h.at[pl.ds(base, BPW)], idx_u)
    pltpu.sync_copy(pid_h.at[pl.ds(base, BPW)], idx_p)
    pltpu.sync_copy(nid_h.at[pl.ds(base, BPW)], idx_n)

    lane = lax.iota(jnp.int32, L)

    for j in range(NCH):

        def fetch(g, carry, j=j):
            boff = pl.multiple_of(j * CH + g * L, L)
            coff = pl.multiple_of(g * L, L)
            vu = idx_u[pl.ds(boff, L)]
            vp = idx_p[pl.ds(boff, L)]
            vn = idx_n[pl.ds(boff, L)]
            for l in range(L):
                i = coff + l
                pltpu.async_copy(
                    ut_h.at[pl.ds(vu[l], 1), :], u_rows.at[pl.ds(i, 1), :], sem)
                pltpu.async_copy(
                    it_h.at[pl.ds(vp[l], 1), :], p_rows.at[pl.ds(i, 1), :], sem)
                pltpu.async_copy(
                    it_h.at[pl.ds(vn[l], 1), :], n_rows.at[pl.ds(i, 1), :], sem)
            return carry

        lax.fori_loop(0, GPC, fetch, 0)

        pltpu.make_async_copy(ut_h.at[pl.ds(0, CH), :], u_rows, sem).wait()
        pltpu.make_async_copy(it_h.at[pl.ds(0, CH), :], p_rows, sem).wait()
        pltpu.make_async_copy(it_h.at[pl.ds(0, CH), :], n_rows, sem).wait()

        def group(g, carry, j=j):
            rvec = lane + g * L
            accp = jnp.zeros((L,), jnp.float32)
            accn = jnp.zeros((L,), jnp.float32)
            for d in range(D):
                cvec = jnp.full((L,), d, jnp.int32)
                u = plsc.load_gather(u_rows, [rvec, cvec])
                p = plsc.load_gather(p_rows, [rvec, cvec])
                n = plsc.load_gather(n_rows, [rvec, cvec])
                accp = accp + u * p
                accn = accn + u * n
            off = pl.multiple_of(j * CH + g * L, L)
            pos_v[pl.ds(off, L)] = accp
            neg_v[pl.ds(off, L)] = accn
            return carry

        lax.fori_loop(0, GPC, group, 0)

    pltpu.sync_copy(pos_v, pos_h.at[pl.ds(base, BPW)])
    pltpu.sync_copy(neg_v, neg_h.at[pl.ds(base, BPW)])


def kernel(user_ids, pos_item_ids, neg_item_ids, user_table, item_table):
    mesh = plsc.VectorSubcoreMesh(core_axis_name="c", subcore_axis_name="s")
    f = functools.partial(
        pl.kernel,
        mesh=mesh,
        compiler_params=pltpu.CompilerParams(needs_layout_passes=False),
        out_type=(
            jax.ShapeDtypeStruct((B,), jnp.float32),
            jax.ShapeDtypeStruct((B,), jnp.float32),
        ),
        scratch_types=[
            pltpu.VMEM((BPW,), jnp.int32),
            pltpu.VMEM((BPW,), jnp.int32),
            pltpu.VMEM((BPW,), jnp.int32),
            pltpu.VMEM((CH, D), jnp.float32),
            pltpu.VMEM((CH, D), jnp.float32),
            pltpu.VMEM((CH, D), jnp.float32),
            pltpu.VMEM((BPW,), jnp.float32),
            pltpu.VMEM((BPW,), jnp.float32),
            pltpu.SemaphoreType.DMA,
        ],
    )(_body)
    return f(
        user_ids.astype(jnp.int32),
        pos_item_ids.astype(jnp.int32),
        neg_item_ids.astype(jnp.int32),
        user_table,
        item_table,
    )
